# baseline (speedup 1.0000x reference)
import functools

import jax
import jax.numpy as jnp
from jax import lax
from jax.experimental import pallas as pl
from jax.experimental.pallas import tpu as pltpu
from jax.experimental.pallas import tpu_sc as plsc

N = 10000
E = 160000
GROUP = 128
NC, NS = 2, 16
NW = NC * NS
IDX_ROWS = 1280
E_PAD = IDX_ROWS * GROUP
RPW = IDX_ROWS // NW
N_PAD = 10112
ROW_BLK = 1000


def _sc_mesh():
    return plsc.VectorSubcoreMesh(core_axis_name="c", subcore_axis_name="s")


_SC_PARAMS = pltpu.CompilerParams(use_tc_tiling_on_sc=False)


NBUF_DEG, NBUF_PLAIN = 4, 4

R0, R1 = 76, 4


def _sc_segment_sum(y, src2d, dst2d, zeros_pad, D, deg_args=None):
    with_deg = deg_args is not None
    nbuf = NBUF_DEG if with_deg else NBUF_PLAIN
    agg_type = jax.ShapeDtypeStruct((NC, N_PAD, D), jnp.float32)
    out_type = [agg_type]
    rmax = max(R0, R1)
    scratch = (
        [pltpu.VMEM((rmax, GROUP), jnp.int32),
         pltpu.VMEM((rmax, GROUP), jnp.int32)]
        + [pltpu.VMEM((GROUP, D), jnp.float32)] * nbuf
        + [pltpu.VMEM_SHARED((N_PAD, D), jnp.float32)]
        + [pltpu.SemaphoreType.DMA] * nbuf
    )
    if with_deg:
        out_type.append(jax.ShapeDtypeStruct((NC, N_PAD, 16), jnp.float32))
        scratch += [pltpu.VMEM((GROUP, 16), jnp.float32),
                    pltpu.VMEM_SHARED((N_PAD, 16), jnp.float32)]
    else:
        out_type = agg_type

    @functools.partial(
        pl.kernel,
        out_type=out_type,
        mesh=_sc_mesh(),
        compiler_params=_SC_PARAMS,
        scratch_types=scratch,
    )
    def agg(*refs):
        if with_deg:
            (y_hbm, src_hbm, dst_hbm, z_hbm, ones_hbm, z16_hbm,
             out_hbm, deg_hbm) = refs[:8]
            rest = refs[8:]
        else:
            y_hbm, src_hbm, dst_hbm, z_hbm, out_hbm = refs[:5]
            rest = refs[5:]
        src_vm, dst_vm = rest[0], rest[1]
        bufs = rest[2:2 + nbuf]
        acc_sh = rest[2 + nbuf]
        sems = rest[3 + nbuf:3 + 2 * nbuf]
        if with_deg:
            ones_vm, dacc_sh = rest[3 + 2 * nbuf], rest[4 + 2 * nbuf]
        c = lax.axis_index("c")
        s = lax.axis_index("s")
        zrows = N_PAD // NS
        pltpu.sync_copy(z_hbm.at[pl.ds(s * zrows, zrows)],
                        acc_sh.at[pl.ds(s * zrows, zrows)])
        if with_deg:
            pltpu.sync_copy(z16_hbm.at[pl.ds(s * zrows, zrows)],
                            dacc_sh.at[pl.ds(s * zrows, zrows)])
            pltpu.sync_copy(ones_hbm, ones_vm)

        def edge_loop(base, rpw):
            pltpu.sync_copy(src_hbm.at[pl.ds(base, rpw)],
                            src_vm.at[pl.ds(0, rpw)])
            pltpu.sync_copy(dst_hbm.at[pl.ds(base, rpw)],
                            dst_vm.at[pl.ds(0, rpw)])
            plsc.subcore_barrier()

            for b in range(nbuf):
                pltpu.async_copy(y_hbm.at[src_vm.at[b]], bufs[b], sems[b])

            @pl.loop(0, rpw, step=nbuf)
            def _(g):
                for b in range(nbuf):
                    @pl.when(g + b < rpw)
                    def _():
                        pltpu.make_async_copy(
                            y_hbm.at[src_vm.at[g + b]], bufs[b],
                            sems[b]).wait()
                        pltpu.sync_copy(bufs[b],
                                        acc_sh.at[dst_vm.at[g + b]],
                                        add=True)
                        if with_deg:
                            pltpu.sync_copy(ones_vm,
                                            dacc_sh.at[dst_vm.at[g + b]],
                                            add=True)

                        @pl.when(g + nbuf + b < rpw)
                        def _():
                            pltpu.async_copy(
                                y_hbm.at[src_vm.at[g + nbuf + b]],
                                bufs[b], sems[b])

        @pl.when(c == 0)
        def _():
            edge_loop(s * R0, R0)

        @pl.when(c != 0)
        def _():
            edge_loop(NS * R0 + s * R1, R1)

        plsc.subcore_barrier()
        pltpu.sync_copy(acc_sh.at[pl.ds(s * zrows, zrows)],
                        out_hbm.at[c, pl.ds(s * zrows, zrows)])
        if with_deg:
            pltpu.sync_copy(dacc_sh.at[pl.ds(s * zrows, zrows)],
                            deg_hbm.at[c, pl.ds(s * zrows, zrows)])

    if with_deg:
        return agg(y, src2d, dst2d, zeros_pad, deg_args[0], deg_args[1])
    return agg(y, src2d, dst2d, zeros_pad)


def _tc_in_proj(x, wcat_t):
    def body(x_ref, w_ref, y1_ref, zr_ref):
        z = jnp.dot(x_ref[...], w_ref[...], preferred_element_type=jnp.float32)
        y1_ref[...] = z[:, :64]
        zr_ref[...] = z[:, 64:]

    return pl.pallas_call(
        body,
        grid=(N // ROW_BLK,),
        in_specs=[
            pl.BlockSpec((ROW_BLK, 256), lambda i: (i, 0)),
            pl.BlockSpec((256, 128), lambda i: (0, 0)),
        ],
        out_specs=[
            pl.BlockSpec((ROW_BLK, 64), lambda i: (i, 0)),
            pl.BlockSpec((ROW_BLK, 64), lambda i: (i, 0)),
        ],
        out_shape=[
            jax.ShapeDtypeStruct((N, 64), jnp.float32),
            jax.ShapeDtypeStruct((N, 64), jnp.float32),
        ],
    )(x, wcat_t)


def _tc_layer1(p, deg, zr, b1):
    def body(p_ref, d_ref, zr_ref, b_ref, h1_ref, inv_ref):
        d = d_ref[0] + d_ref[1]
        inv = 1.0 / jnp.maximum(d, 1.0)
        col = lax.broadcasted_iota(jnp.int32, inv.shape, 1)
        inv = jnp.where(col == 1, jnp.minimum(d[:, 0:1], 1.0), inv)
        inv_ref[...] = inv
        m = (p_ref[0] + p_ref[1]) * inv[:, 0:1]
        h1_ref[...] = jnp.maximum(m + zr_ref[...] + b_ref[...], 0.0)

    return pl.pallas_call(
        body,
        grid=(N // ROW_BLK,),
        in_specs=[
            pl.BlockSpec((2, ROW_BLK, 64), lambda i: (0, i, 0)),
            pl.BlockSpec((2, ROW_BLK, 16), lambda i: (0, i, 0)),
            pl.BlockSpec((ROW_BLK, 64), lambda i: (i, 0)),
            pl.BlockSpec((1, 64), lambda i: (0, 0)),
        ],
        out_specs=[
            pl.BlockSpec((ROW_BLK, 64), lambda i: (i, 0)),
            pl.BlockSpec((ROW_BLK, 16), lambda i: (i, 0)),
        ],
        out_shape=[
            jax.ShapeDtypeStruct((N, 64), jnp.float32),
            jax.ShapeDtypeStruct((N, 16), jnp.float32),
        ],
    )(p, deg, zr, b1)


def _tc_layer2(q, inv, h1, w2l_t, w2r_t, b2):
    def body(q_ref, inv_ref, h1_ref, wl_ref, wr_ref, b_ref, h2_ref, m1_ref):
        m1 = (q_ref[0] + q_ref[1]) * inv_ref[:, 0:1]
        m1_ref[...] = m1
        h2_ref[...] = (
            jnp.dot(m1, wl_ref[...], preferred_element_type=jnp.float32)
            + jnp.dot(h1_ref[...], wr_ref[...], preferred_element_type=jnp.float32)
            + b_ref[...]
        )

    return pl.pallas_call(
        body,
        grid=(N // ROW_BLK,),
        in_specs=[
            pl.BlockSpec((2, ROW_BLK, 64), lambda i: (0, i, 0)),
            pl.BlockSpec((ROW_BLK, 16), lambda i: (i, 0)),
            pl.BlockSpec((ROW_BLK, 64), lambda i: (i, 0)),
            pl.BlockSpec((64, 128), lambda i: (0, 0)),
            pl.BlockSpec((64, 128), lambda i: (0, 0)),
            pl.BlockSpec((1, 128), lambda i: (0, 0)),
        ],
        out_specs=[
            pl.BlockSpec((ROW_BLK, 128), lambda i: (i, 0)),
            pl.BlockSpec((ROW_BLK, 64), lambda i: (i, 0)),
        ],
        out_shape=[
            jax.ShapeDtypeStruct((N, 128), jnp.float32),
            jax.ShapeDtypeStruct((N, 64), jnp.float32),
        ],
    )(q, inv, h1, w2l_t, w2r_t, b2)


def _tc_layer3(r, inv, m1, h2, a3_t, b3_t, c3, w3r_t, b3, wlin_t, blin):
    def body(r_ref, inv_ref, m1_ref, h2_ref, wa_ref, wb_ref, c3_ref,
             wr_ref, b_ref, wo_ref, bo_ref, out_ref):
        m2 = (r_ref[0] + r_ref[1]) * inv_ref[:, 0:1]
        gate = inv_ref[:, 1:2]
        pre = (
            jnp.dot(m2, wa_ref[...], preferred_element_type=jnp.float32)
            + jnp.dot(m1_ref[...], wb_ref[...], preferred_element_type=jnp.float32)
            + jnp.dot(h2_ref[...], wr_ref[...], preferred_element_type=jnp.float32)
            + gate * c3_ref[...]
            + b_ref[...]
        )
        h3 = jnp.maximum(pre, 0.0)
        logits = jnp.dot(h3, wo_ref[...], preferred_element_type=jnp.float32)
        logits = logits + bo_ref[...]
        mx = jnp.max(logits, axis=1, keepdims=True)
        e = jnp.exp(logits - mx)
        out_ref[...] = e / jnp.sum(e, axis=1, keepdims=True)

    return pl.pallas_call(
        body,
        grid=(N // ROW_BLK,),
        in_specs=[
            pl.BlockSpec((2, ROW_BLK, 64), lambda i: (0, i, 0)),
            pl.BlockSpec((ROW_BLK, 16), lambda i: (i, 0)),
            pl.BlockSpec((ROW_BLK, 64), lambda i: (i, 0)),
            pl.BlockSpec((ROW_BLK, 128), lambda i: (i, 0)),
            pl.BlockSpec((64, 256), lambda i: (0, 0)),
            pl.BlockSpec((64, 256), lambda i: (0, 0)),
            pl.BlockSpec((1, 256), lambda i: (0, 0)),
            pl.BlockSpec((128, 256), lambda i: (0, 0)),
            pl.BlockSpec((1, 256), lambda i: (0, 0)),
            pl.BlockSpec((256, 64), lambda i: (0, 0)),
            pl.BlockSpec((1, 64), lambda i: (0, 0)),
        ],
        out_specs=pl.BlockSpec((ROW_BLK, 64), lambda i: (i, 0)),
        out_shape=jax.ShapeDtypeStruct((N, 64), jnp.float32),
    )(r, inv, m1, h2, a3_t, b3_t, c3, w3r_t, b3, wlin_t, blin)


def kernel(x, edge_index, W1l, W1r, b1, W2l, W2r, b2, W3l, W3r, b3,
           Wlin, blin):
    src = edge_index[0].astype(jnp.int32)
    dst = edge_index[1].astype(jnp.int32)
    npad = E_PAD - E
    src2d = jnp.concatenate(
        [src, jnp.zeros((npad,), jnp.int32)]).reshape(IDX_ROWS, GROUP)
    dst2d = jnp.concatenate(
        [dst, jnp.full((npad,), N, jnp.int32)]).reshape(IDX_ROWS, GROUP)

    z64 = jnp.zeros((N_PAD, 64), jnp.float32)
    z16 = jnp.zeros((N_PAD, 16), jnp.float32)
    ones_rows = jnp.zeros((GROUP, 16), jnp.float32).at[:, 0].set(1.0)

    wcat_t = jnp.concatenate([W1l, W1r], axis=0).T
    a3_t = (W3l @ W2l).T
    b3f_t = (W3l @ W2r).T
    c3 = (W3l @ b2).reshape(1, 256)

    y1, zr = _tc_in_proj(x, wcat_t)
    p, deg = _sc_segment_sum(y1, src2d, dst2d, z64, 64,
                             deg_args=(ones_rows, z16))
    h1, inv = _tc_layer1(p, deg, zr, b1.reshape(1, 64))
    q = _sc_segment_sum(h1, src2d, dst2d, z64, 64)
    h2, m1 = _tc_layer2(q, inv, h1, W2l.T, W2r.T, b2.reshape(1, 128))
    r = _sc_segment_sum(m1, src2d, dst2d, z64, 64)
    out = _tc_layer3(r, inv, m1, h2, a3_t, b3f_t, c3, W3r.T,
                     b3.reshape(1, 256), Wlin.T, blin.reshape(1, 64))
    return out


# speedup vs baseline: 1.0088x; 1.0088x over previous
import functools

import jax
import jax.numpy as jnp
from jax import lax
from jax.experimental import pallas as pl
from jax.experimental.pallas import tpu as pltpu
from jax.experimental.pallas import tpu_sc as plsc

N = 10000
E = 160000
GROUP = 128
NC, NS = 2, 16
NW = NC * NS
IDX_ROWS = 1280
E_PAD = IDX_ROWS * GROUP
RPW = IDX_ROWS // NW
N_PAD = 10112
ROW_BLK = 1000


def _sc_mesh():
    return plsc.VectorSubcoreMesh(core_axis_name="c", subcore_axis_name="s")


_SC_PARAMS = pltpu.CompilerParams(use_tc_tiling_on_sc=False)


NBUF_DEG, NBUF_PLAIN = 4, 4

R0, R1 = 72, 8


def _sc_segment_sum(y, src2d, dst2d, zeros_pad, D, deg_args=None):
    with_deg = deg_args is not None
    nbuf = NBUF_DEG if with_deg else NBUF_PLAIN
    agg_type = jax.ShapeDtypeStruct((NC, N_PAD, D), jnp.float32)
    out_type = [agg_type]
    rmax = max(R0, R1)
    scratch = (
        [pltpu.VMEM((rmax, GROUP), jnp.int32),
         pltpu.VMEM((rmax, GROUP), jnp.int32)]
        + [pltpu.VMEM((GROUP, D), jnp.float32)] * nbuf
        + [pltpu.VMEM_SHARED((N_PAD, D), jnp.float32)]
        + [pltpu.SemaphoreType.DMA] * nbuf
    )
    if with_deg:
        out_type.append(jax.ShapeDtypeStruct((NC, N_PAD, 16), jnp.float32))
        scratch += [pltpu.VMEM((GROUP, 16), jnp.float32),
                    pltpu.VMEM_SHARED((N_PAD, 16), jnp.float32)]
    else:
        out_type = agg_type

    @functools.partial(
        pl.kernel,
        out_type=out_type,
        mesh=_sc_mesh(),
        compiler_params=_SC_PARAMS,
        scratch_types=scratch,
    )
    def agg(*refs):
        if with_deg:
            (y_hbm, src_hbm, dst_hbm, z_hbm, ones_hbm, z16_hbm,
             out_hbm, deg_hbm) = refs[:8]
            rest = refs[8:]
        else:
            y_hbm, src_hbm, dst_hbm, z_hbm, out_hbm = refs[:5]
            rest = refs[5:]
        src_vm, dst_vm = rest[0], rest[1]
        bufs = rest[2:2 + nbuf]
        acc_sh = rest[2 + nbuf]
        sems = rest[3 + nbuf:3 + 2 * nbuf]
        if with_deg:
            ones_vm, dacc_sh = rest[3 + 2 * nbuf], rest[4 + 2 * nbuf]
        c = lax.axis_index("c")
        s = lax.axis_index("s")
        zrows = N_PAD // NS
        pltpu.sync_copy(z_hbm.at[pl.ds(s * zrows, zrows)],
                        acc_sh.at[pl.ds(s * zrows, zrows)])
        if with_deg:
            pltpu.sync_copy(z16_hbm.at[pl.ds(s * zrows, zrows)],
                            dacc_sh.at[pl.ds(s * zrows, zrows)])
            pltpu.sync_copy(ones_hbm, ones_vm)

        def edge_loop(base, rpw):
            pltpu.sync_copy(src_hbm.at[pl.ds(base, rpw)],
                            src_vm.at[pl.ds(0, rpw)])
            pltpu.sync_copy(dst_hbm.at[pl.ds(base, rpw)],
                            dst_vm.at[pl.ds(0, rpw)])
            plsc.subcore_barrier()

            for b in range(nbuf):
                pltpu.async_copy(y_hbm.at[src_vm.at[b]], bufs[b], sems[b])

            @pl.loop(0, rpw, step=nbuf)
            def _(g):
                for b in range(nbuf):
                    @pl.when(g + b < rpw)
                    def _():
                        pltpu.make_async_copy(
                            y_hbm.at[src_vm.at[g + b]], bufs[b],
                            sems[b]).wait()
                        pltpu.sync_copy(bufs[b],
                                        acc_sh.at[dst_vm.at[g + b]],
                                        add=True)
                        if with_deg:
                            pltpu.sync_copy(ones_vm,
                                            dacc_sh.at[dst_vm.at[g + b]],
                                            add=True)

                        @pl.when(g + nbuf + b < rpw)
                        def _():
                            pltpu.async_copy(
                                y_hbm.at[src_vm.at[g + nbuf + b]],
                                bufs[b], sems[b])

        @pl.when(c == 0)
        def _():
            edge_loop(s * R0, R0)

        @pl.when(c != 0)
        def _():
            edge_loop(NS * R0 + s * R1, R1)

        plsc.subcore_barrier()
        pltpu.sync_copy(acc_sh.at[pl.ds(s * zrows, zrows)],
                        out_hbm.at[c, pl.ds(s * zrows, zrows)])
        if with_deg:
            pltpu.sync_copy(dacc_sh.at[pl.ds(s * zrows, zrows)],
                            deg_hbm.at[c, pl.ds(s * zrows, zrows)])

    if with_deg:
        return agg(y, src2d, dst2d, zeros_pad, deg_args[0], deg_args[1])
    return agg(y, src2d, dst2d, zeros_pad)


def _tc_in_proj(x, wcat_t):
    def body(x_ref, w_ref, y1_ref, zr_ref):
        z = jnp.dot(x_ref[...], w_ref[...], preferred_element_type=jnp.float32)
        y1_ref[...] = z[:, :64]
        zr_ref[...] = z[:, 64:]

    return pl.pallas_call(
        body,
        grid=(N // ROW_BLK,),
        in_specs=[
            pl.BlockSpec((ROW_BLK, 256), lambda i: (i, 0)),
            pl.BlockSpec((256, 128), lambda i: (0, 0)),
        ],
        out_specs=[
            pl.BlockSpec((ROW_BLK, 64), lambda i: (i, 0)),
            pl.BlockSpec((ROW_BLK, 64), lambda i: (i, 0)),
        ],
        out_shape=[
            jax.ShapeDtypeStruct((N, 64), jnp.float32),
            jax.ShapeDtypeStruct((N, 64), jnp.float32),
        ],
    )(x, wcat_t)


def _tc_layer1(p, deg, zr, b1):
    def body(p_ref, d_ref, zr_ref, b_ref, h1_ref, inv_ref):
        d = d_ref[0] + d_ref[1]
        inv = 1.0 / jnp.maximum(d, 1.0)
        col = lax.broadcasted_iota(jnp.int32, inv.shape, 1)
        inv = jnp.where(col == 1, jnp.minimum(d[:, 0:1], 1.0), inv)
        inv_ref[...] = inv
        m = (p_ref[0] + p_ref[1]) * inv[:, 0:1]
        h1_ref[...] = jnp.maximum(m + zr_ref[...] + b_ref[...], 0.0)

    return pl.pallas_call(
        body,
        grid=(N // ROW_BLK,),
        in_specs=[
            pl.BlockSpec((2, ROW_BLK, 64), lambda i: (0, i, 0)),
            pl.BlockSpec((2, ROW_BLK, 16), lambda i: (0, i, 0)),
            pl.BlockSpec((ROW_BLK, 64), lambda i: (i, 0)),
            pl.BlockSpec((1, 64), lambda i: (0, 0)),
        ],
        out_specs=[
            pl.BlockSpec((ROW_BLK, 64), lambda i: (i, 0)),
            pl.BlockSpec((ROW_BLK, 16), lambda i: (i, 0)),
        ],
        out_shape=[
            jax.ShapeDtypeStruct((N, 64), jnp.float32),
            jax.ShapeDtypeStruct((N, 16), jnp.float32),
        ],
    )(p, deg, zr, b1)


def _tc_layer2(q, inv, h1, w2l_t, w2r_t, b2):
    def body(q_ref, inv_ref, h1_ref, wl_ref, wr_ref, b_ref, h2_ref, m1_ref):
        m1 = (q_ref[0] + q_ref[1]) * inv_ref[:, 0:1]
        m1_ref[...] = m1
        h2_ref[...] = (
            jnp.dot(m1, wl_ref[...], preferred_element_type=jnp.float32)
            + jnp.dot(h1_ref[...], wr_ref[...], preferred_element_type=jnp.float32)
            + b_ref[...]
        )

    return pl.pallas_call(
        body,
        grid=(N // ROW_BLK,),
        in_specs=[
            pl.BlockSpec((2, ROW_BLK, 64), lambda i: (0, i, 0)),
            pl.BlockSpec((ROW_BLK, 16), lambda i: (i, 0)),
            pl.BlockSpec((ROW_BLK, 64), lambda i: (i, 0)),
            pl.BlockSpec((64, 128), lambda i: (0, 0)),
            pl.BlockSpec((64, 128), lambda i: (0, 0)),
            pl.BlockSpec((1, 128), lambda i: (0, 0)),
        ],
        out_specs=[
            pl.BlockSpec((ROW_BLK, 128), lambda i: (i, 0)),
            pl.BlockSpec((ROW_BLK, 64), lambda i: (i, 0)),
        ],
        out_shape=[
            jax.ShapeDtypeStruct((N, 128), jnp.float32),
            jax.ShapeDtypeStruct((N, 64), jnp.float32),
        ],
    )(q, inv, h1, w2l_t, w2r_t, b2)


def _tc_layer3(r, inv, m1, h2, a3_t, b3_t, c3, w3r_t, b3, wlin_t, blin):
    def body(r_ref, inv_ref, m1_ref, h2_ref, wa_ref, wb_ref, c3_ref,
             wr_ref, b_ref, wo_ref, bo_ref, out_ref):
        m2 = (r_ref[0] + r_ref[1]) * inv_ref[:, 0:1]
        gate = inv_ref[:, 1:2]
        pre = (
            jnp.dot(m2, wa_ref[...], preferred_element_type=jnp.float32)
            + jnp.dot(m1_ref[...], wb_ref[...], preferred_element_type=jnp.float32)
            + jnp.dot(h2_ref[...], wr_ref[...], preferred_element_type=jnp.float32)
            + gate * c3_ref[...]
            + b_ref[...]
        )
        h3 = jnp.maximum(pre, 0.0)
        logits = jnp.dot(h3, wo_ref[...], preferred_element_type=jnp.float32)
        logits = logits + bo_ref[...]
        mx = jnp.max(logits, axis=1, keepdims=True)
        e = jnp.exp(logits - mx)
        out_ref[...] = e / jnp.sum(e, axis=1, keepdims=True)

    return pl.pallas_call(
        body,
        grid=(N // ROW_BLK,),
        in_specs=[
            pl.BlockSpec((2, ROW_BLK, 64), lambda i: (0, i, 0)),
            pl.BlockSpec((ROW_BLK, 16), lambda i: (i, 0)),
            pl.BlockSpec((ROW_BLK, 64), lambda i: (i, 0)),
            pl.BlockSpec((ROW_BLK, 128), lambda i: (i, 0)),
            pl.BlockSpec((64, 256), lambda i: (0, 0)),
            pl.BlockSpec((64, 256), lambda i: (0, 0)),
            pl.BlockSpec((1, 256), lambda i: (0, 0)),
            pl.BlockSpec((128, 256), lambda i: (0, 0)),
            pl.BlockSpec((1, 256), lambda i: (0, 0)),
            pl.BlockSpec((256, 64), lambda i: (0, 0)),
            pl.BlockSpec((1, 64), lambda i: (0, 0)),
        ],
        out_specs=pl.BlockSpec((ROW_BLK, 64), lambda i: (i, 0)),
        out_shape=jax.ShapeDtypeStruct((N, 64), jnp.float32),
    )(r, inv, m1, h2, a3_t, b3_t, c3, w3r_t, b3, wlin_t, blin)


def kernel(x, edge_index, W1l, W1r, b1, W2l, W2r, b2, W3l, W3r, b3,
           Wlin, blin):
    src = edge_index[0].astype(jnp.int32)
    dst = edge_index[1].astype(jnp.int32)
    npad = E_PAD - E
    src2d = jnp.concatenate(
        [src, jnp.zeros((npad,), jnp.int32)]).reshape(IDX_ROWS, GROUP)
    dst2d = jnp.concatenate(
        [dst, jnp.full((npad,), N, jnp.int32)]).reshape(IDX_ROWS, GROUP)

    z64 = jnp.zeros((N_PAD, 64), jnp.float32)
    z16 = jnp.zeros((N_PAD, 16), jnp.float32)
    ones_rows = jnp.zeros((GROUP, 16), jnp.float32).at[:, 0].set(1.0)

    wcat_t = jnp.concatenate([W1l, W1r], axis=0).T
    a3_t = (W3l @ W2l).T
    b3f_t = (W3l @ W2r).T
    c3 = (W3l @ b2).reshape(1, 256)

    y1, zr = _tc_in_proj(x, wcat_t)
    p, deg = _sc_segment_sum(y1, src2d, dst2d, z64, 64,
                             deg_args=(ones_rows, z16))
    h1, inv = _tc_layer1(p, deg, zr, b1.reshape(1, 64))
    q = _sc_segment_sum(h1, src2d, dst2d, z64, 64)
    h2, m1 = _tc_layer2(q, inv, h1, W2l.T, W2r.T, b2.reshape(1, 128))
    r = _sc_segment_sum(m1, src2d, dst2d, z64, 64)
    out = _tc_layer3(r, inv, m1, h2, a3_t, b3f_t, c3, W3r.T,
                     b3.reshape(1, 256), Wlin.T, blin.reshape(1, 64))
    return out
